# baseline (speedup 1.0000x reference)
import functools

import jax
import jax.numpy as jnp
from jax import lax
from jax.experimental import pallas as pl
from jax.experimental.pallas import tpu as pltpu
from jax.experimental.pallas import tpu_sc as plsc


@functools.partial(jax.jit, static_argnames=("n_rows", "dim"))
def _sc_gather(x_flat, table, n_rows, dim):
    info = plsc.get_sparse_core_info()
    nc, ns, nl = info.num_cores, info.num_subcores, info.num_lanes
    nw = nc * ns

    g4 = 128 // dim
    n128 = n_rows // g4
    b_per_w = n128 // nw
    r_per_w = b_per_w * g4
    chunk = 256
    n_chunks = b_per_w // chunk

    mesh = plsc.VectorSubcoreMesh(core_axis_name="c", subcore_axis_name="s")

    @functools.partial(
        pl.kernel,
        mesh=mesh,
        out_type=jax.ShapeDtypeStruct((n128, 128), jnp.float32),
        scratch_types=[
            pltpu.VMEM((r_per_w,), jnp.int32),
            pltpu.VMEM((g4, b_per_w), jnp.int32),
            pltpu.VMEM((2, g4, chunk, dim), jnp.float32),
            pltpu.SemaphoreType.DMA,
            pltpu.SemaphoreType.DMA,
            pltpu.SemaphoreType.DMA,
            pltpu.SemaphoreType.DMA,
        ],
        compiler_params=pltpu.CompilerParams(
            use_tc_tiling_on_sc=False, needs_layout_passes=False),
    )
    def k(xh, tab_hbm, out_hbm, raw_v, idx_v, rows_v, g0, g1, w0, w1):
        wid = lax.axis_index("s") * nc + lax.axis_index("c")
        base = wid * b_per_w
        gsem = (g0, g1)
        wsem = (w0, w1)

        pltpu.sync_copy(xh.at[pl.ds(base * g4, r_per_w)], raw_v)
        lanes4 = lax.iota(jnp.int32, nl) * g4

        def deint(t, g):
            src = lanes4 + (t * (nl * g4) + g)
            idx_v[g, pl.ds(t * nl, nl)] = plsc.load_gather(raw_v, [src])
            return g

        for g in range(g4):
            lax.fori_loop(0, b_per_w // nl, deint, g)

        def gather(i):
            s = i % 2
            last = None
            for g in range(g4):
                last = pltpu.async_copy(
                    tab_hbm.at[idx_v.at[g, pl.ds(i * chunk, chunk)]],
                    rows_v.at[s, g],
                    gsem[s])
            return last

        def writeback(i):
            s = i % 2
            last = None
            for g in range(g4):
                last = pltpu.async_copy(
                    rows_v.at[s, g],
                    out_hbm.at[pl.ds(base + i * chunk, chunk),
                               pl.ds(g * dim, dim)],
                    wsem[s])
            return last

        def wait4(cp):
            for _ in range(g4):
                cp.wait()

        gathers = [None] * n_chunks
        writes = [None] * n_chunks
        gathers[0] = gather(0)
        for i in range(n_chunks):
            if i + 1 < n_chunks:
                if i >= 1:
                    wait4(writes[i - 1])
                gathers[i + 1] = gather(i + 1)
            wait4(gathers[i])
            writes[i] = writeback(i)
        if n_chunks >= 2:
            wait4(writes[n_chunks - 2])
        wait4(writes[n_chunks - 1])

    return k(x_flat, table)


def kernel(x, table):
    b, s = x.shape
    dim = table.shape[1]
    n_rows = b * s
    out = _sc_gather(x.reshape(n_rows).astype(jnp.int32), table, n_rows, dim)
    return out.reshape(b, s, dim)


# speedup vs baseline: 1.0049x; 1.0049x over previous
import functools

import jax
import jax.numpy as jnp
from jax import lax
from jax.experimental import pallas as pl
from jax.experimental.pallas import tpu as pltpu
from jax.experimental.pallas import tpu_sc as plsc


@functools.partial(jax.jit, static_argnames=("b", "s", "dim"))
def _sc_gather(x_flat, table, b, s, dim):
    info = plsc.get_sparse_core_info()
    nc, ns = info.num_cores, info.num_subcores
    nw = nc * ns

    n_rows = b * s
    r_per_w = n_rows // nw
    x_per_w = b // nw
    xr_chunk = 32
    chunk = xr_chunk * s
    n_chunks = x_per_w // xr_chunk

    mesh = plsc.VectorSubcoreMesh(core_axis_name="c", subcore_axis_name="s")

    @functools.partial(
        pl.kernel,
        mesh=mesh,
        out_type=jax.ShapeDtypeStruct((b, s, dim), jnp.float32),
        scratch_types=[
            pltpu.VMEM((r_per_w,), jnp.int32),
            pltpu.VMEM((2, chunk, dim), jnp.float32),
            pltpu.SemaphoreType.DMA,
            pltpu.SemaphoreType.DMA,
            pltpu.SemaphoreType.DMA,
            pltpu.SemaphoreType.DMA,
        ],
        compiler_params=pltpu.CompilerParams(use_tc_tiling_on_sc=False),
    )
    def k(xh, tab_hbm, out_hbm, raw_v, rows_v, g0, g1, w0, w1):
        wid = lax.axis_index("s") * nc + lax.axis_index("c")
        base = wid * r_per_w
        xbase = wid * x_per_w
        gsem = (g0, g1)
        wsem = (w0, w1)

        pltpu.sync_copy(xh.at[pl.ds(base, r_per_w)], raw_v)

        def gather(i):
            sl = i % 2
            return pltpu.async_copy(
                tab_hbm.at[raw_v.at[pl.ds(i * chunk, chunk)]],
                rows_v.at[sl],
                gsem[sl])

        def writeback(i):
            sl = i % 2
            last = None
            for r in range(xr_chunk):
                last = pltpu.async_copy(
                    rows_v.at[sl, pl.ds(r * s, s)],
                    out_hbm.at[xbase + i * xr_chunk + r],
                    wsem[sl])
            return last

        def wait_writes(cp):
            for _ in range(xr_chunk):
                cp.wait()

        gathers = [None] * n_chunks
        writes = [None] * n_chunks
        gathers[0] = gather(0)
        for i in range(n_chunks):
            if i + 1 < n_chunks:
                if i >= 1:
                    wait_writes(writes[i - 1])
                gathers[i + 1] = gather(i + 1)
            gathers[i].wait()
            writes[i] = writeback(i)
        if n_chunks >= 2:
            wait_writes(writes[n_chunks - 2])
        wait_writes(writes[n_chunks - 1])

    return k(x_flat, table)


def kernel(x, table):
    b, s = x.shape
    dim = table.shape[1]
    out = _sc_gather(x.reshape(b * s).astype(jnp.int32), table, b, s, dim)
    return out
